# baseline (speedup 1.0000x reference)
import jax
import jax.numpy as jnp
from jax import lax
from jax.experimental import pallas as pl


def _gnn_fused_kernel(h_ref, wmsg_ref, bmsg_ref, wih_ref, whh_ref,
                      bih_ref, bhh_ref, out_ref):
    h = h_ref[...]
    n = jnp.float32(h.shape[0])
    f = h.shape[1]
    wmsg = wmsg_ref[...]
    a = wmsg[:, :f]
    b = wmsg[:, f:]
    wih = wih_ref[...]
    whh = whh_ref[...]

    def dotT(x, w):
        return lax.dot_general(x, w, (((1,), (1,)), ((), ())),
                               preferred_element_type=jnp.float32)

    col_sum = jnp.sum(h, axis=0, keepdims=True)
    base = dotT(col_sum, a) + n * bmsg_ref[...]
    agg = n * dotT(h, b) + base

    gi = dotT(agg, wih) + bih_ref[...]
    gh = dotT(h, whh) + bhh_ref[...]
    rz = jax.nn.sigmoid(gi[:, :2 * f] + gh[:, :2 * f])
    r = rz[:, :f]
    z = rz[:, f:]
    ng = jnp.tanh(gi[:, 2 * f:] + r * gh[:, 2 * f:])
    out_ref[...] = (1.0 - z) * ng + z * h


def kernel(h, W_msg, b_msg, W_ih, W_hh, b_ih, b_hh):
    return pl.pallas_call(
        _gnn_fused_kernel,
        out_shape=jax.ShapeDtypeStruct(h.shape, h.dtype),
    )(h, W_msg, b_msg.reshape(1, -1), W_ih, W_hh,
      b_ih.reshape(1, -1), b_hh.reshape(1, -1))


# speedup vs baseline: 1.0957x; 1.0957x over previous
import jax
import jax.numpy as jnp
from jax.experimental import pallas as pl


def _gnn_fused_kernel(h_ref, wsrc_ref, wdst_ref, bmsg_ref,
                      wih_ref, whh_ref, bih_ref, bhh_ref, out_ref):
    h = h_ref[...]
    n = jnp.float32(h.shape[0])
    f = h.shape[1]

    col_sum = jnp.sum(h, axis=0, keepdims=True)
    base = (jnp.dot(col_sum, wsrc_ref[...], preferred_element_type=jnp.float32)
            + n * bmsg_ref[...])
    agg = n * jnp.dot(h, wdst_ref[...], preferred_element_type=jnp.float32) + base

    gi = jnp.dot(agg, wih_ref[...], preferred_element_type=jnp.float32) + bih_ref[...]
    gh = jnp.dot(h, whh_ref[...], preferred_element_type=jnp.float32) + bhh_ref[...]
    rz = jax.nn.sigmoid(gi[:, :2 * f] + gh[:, :2 * f])
    r = rz[:, :f]
    z = rz[:, f:]
    ng = jnp.tanh(gi[:, 2 * f:] + r * gh[:, 2 * f:])
    out_ref[...] = (1.0 - z) * ng + z * h


def kernel(h, W_msg, b_msg, W_ih, W_hh, b_ih, b_hh):
    f = h.shape[1]
    wsrc = W_msg[:, :f].T
    wdst = W_msg[:, f:].T
    wih = W_ih.T
    whh = W_hh.T
    bmsg = b_msg.reshape(1, -1)
    bih = b_ih.reshape(1, -1)
    bhh = b_hh.reshape(1, -1)
    return pl.pallas_call(
        _gnn_fused_kernel,
        out_shape=jax.ShapeDtypeStruct(h.shape, h.dtype),
    )(h, wsrc, wdst, bmsg, wih, whh, bih, bhh)
